# baseline (speedup 1.0000x reference)
import jax
import jax.numpy as jnp
from jax.experimental import pallas as pl
from jax.experimental.pallas import tpu as pltpu

N = 100000
D = 128
BLK = 5000

_DN = (((1,), (1,)), ((), ()))


def _body(h0i, h1i, h0c, h1c, w1, b1, w3, b3, oi, oc):
    j = pl.program_id(1)

    @pl.when(j == 0)
    def _():
        w1f = w1[:].astype(jnp.bfloat16)
        a0 = h0i[:]
        a1 = h1i[:]
        g = jax.nn.sigmoid(
            jax.lax.dot_general(a0.astype(jnp.bfloat16), w1f[:, :D], _DN,
                                preferred_element_type=jnp.float32)
            + jax.lax.dot_general(a1.astype(jnp.bfloat16), w1f[:, D:], _DN,
                                  preferred_element_type=jnp.float32)
            + b1[:]
        )
        oi[:] = a1 + g * (a0 - a1)

    @pl.when(j == 1)
    def _():
        w3f = w3[:].astype(jnp.bfloat16)
        c0 = h0c[:]
        c1 = h1c[:]
        g2 = jax.nn.sigmoid(
            jax.lax.dot_general(c0.astype(jnp.bfloat16), w3f[:, :D], _DN,
                                preferred_element_type=jnp.float32)
            + jax.lax.dot_general(c1.astype(jnp.bfloat16), w3f[:, D:], _DN,
                                  preferred_element_type=jnp.float32)
            + b3[:]
        )
        oc[:] = c1 + g2 * (c0 - c1)


@jax.jit
def kernel(h0_i, h0_c, h1_i, h1_c, Wg1, bg1, Wg3, bg3):
    b1 = bg1.reshape(1, D)
    b3 = bg3.reshape(1, D)

    row_spec = pl.BlockSpec((BLK, D), lambda i, j: (i, 0))
    w_spec = pl.BlockSpec((D, 2 * D), lambda i, j: (0, 0))
    b_spec = pl.BlockSpec((1, D), lambda i, j: (0, 0))

    out_shape = (
        jax.ShapeDtypeStruct((N, D), jnp.float32),
        jax.ShapeDtypeStruct((N, D), jnp.float32),
    )
    oi, oc = pl.pallas_call(
        _body,
        grid=(N // BLK, 2),
        in_specs=[
            row_spec,
            row_spec,
            row_spec,
            row_spec,
            w_spec,
            b_spec,
            w_spec,
            b_spec,
        ],
        out_specs=(row_spec, row_spec),
        out_shape=out_shape,
        compiler_params=pltpu.CompilerParams(
            dimension_semantics=("arbitrary", "arbitrary"),
        ),
    )(h0_i, h1_i, h0_c, h1_c, Wg1, b1, Wg3, b3)
    return (oi, oc)


# speedup vs baseline: 1.4712x; 1.4712x over previous
import jax
import jax.numpy as jnp
from jax.experimental import pallas as pl
from jax.experimental.pallas import tpu as pltpu

N = 100000
D = 128
BLK = 5000

_DN = (((1,), (1,)), ((), ()))


def _body(h0i, h1i, h0c, h1c, w1, b1, w3, b3, oi, oc):
    w1f = w1[:].astype(jnp.bfloat16)
    w3f = w3[:].astype(jnp.bfloat16)
    a0 = h0i[:]
    a1 = h1i[:]
    g = jax.nn.sigmoid(
        jax.lax.dot_general(a0.astype(jnp.bfloat16), w1f[:, :D], _DN,
                            preferred_element_type=jnp.float32)
        + jax.lax.dot_general(a1.astype(jnp.bfloat16), w1f[:, D:], _DN,
                              preferred_element_type=jnp.float32)
        + b1[:]
    )
    oi[:] = a1 + g * (a0 - a1)
    c0 = h0c[:]
    c1 = h1c[:]
    g2 = jax.nn.sigmoid(
        jax.lax.dot_general(c0.astype(jnp.bfloat16), w3f[:, :D], _DN,
                            preferred_element_type=jnp.float32)
        + jax.lax.dot_general(c1.astype(jnp.bfloat16), w3f[:, D:], _DN,
                              preferred_element_type=jnp.float32)
        + b3[:]
    )
    oc[:] = c1 + g2 * (c0 - c1)


@jax.jit
def kernel(h0_i, h0_c, h1_i, h1_c, Wg1, bg1, Wg3, bg3):
    b1 = bg1.reshape(1, D)
    b3 = bg3.reshape(1, D)

    row_spec = pl.BlockSpec((BLK, D), lambda i: (i, 0))
    w_spec = pl.BlockSpec((D, 2 * D), lambda i: (0, 0))
    b_spec = pl.BlockSpec((1, D), lambda i: (0, 0))

    out_shape = (
        jax.ShapeDtypeStruct((N, D), jnp.float32),
        jax.ShapeDtypeStruct((N, D), jnp.float32),
    )
    oi, oc = pl.pallas_call(
        _body,
        grid=(N // BLK,),
        in_specs=[
            row_spec,
            row_spec,
            row_spec,
            row_spec,
            w_spec,
            b_spec,
            w_spec,
            b_spec,
        ],
        out_specs=(row_spec, row_spec),
        out_shape=out_shape,
        compiler_params=pltpu.CompilerParams(
            dimension_semantics=("arbitrary",),
        ),
    )(h0_i, h1_i, h0_c, h1_c, Wg1, b1, Wg3, b3)
    return (oi, oc)
